# baseline (speedup 1.0000x reference)
import functools

import jax
import jax.numpy as jnp
from jax import lax
from jax.experimental import pallas as pl
from jax.experimental.pallas import tpu as pltpu
from jax.experimental.pallas import tpu_sc as plsc

B, L = 1024, 200
ITEM_DIM, CATE_DIM = 128, 64
WEEK_DIM, HOUR_DIM, BEH_DIM = 3, 5, 8
SMALL_D = WEEK_DIM + HOUR_DIM + BEH_DIM + 3
OUT_D = ITEM_DIM + CATE_DIM + SMALL_D

NW = 32
TOTAL = B * L
PER_W = TOTAL // NW
CH = 32
K = PER_W // CH
NSLOT = 4
NIMG = 2

_SM_OFF = ITEM_DIM + CATE_DIM
_W_OFF = 0
_H_OFF = WEEK_DIM
_B_OFF = WEEK_DIM + HOUR_DIM
_S_OFF = _B_OFF + BEH_DIM


def _sc_body(items_h, cates_h, wk_h, hr_h, bh_h, wkend_h, days_h, dte_h,
             itab_h, ctab_h, wtab_h, htab_h, btab_h,
             out_h,
             idx_i, idx_c, idx_w, idx_hr, idx_b,
             sc_wkend, sc_days, sc_dte,
             wtab_v, htab_v, btab_v,
             item_b, cate_b, sm_b, stage,
             *sems):
  sid = lax.axis_index("s")
  wid = sid * 2 + lax.axis_index("c")
  sem_i = sems[0:NSLOT]
  sem_c = sems[NSLOT:2 * NSLOT]
  sem_a = sems[2 * NSLOT:3 * NSLOT]
  sem_s = sems[3 * NSLOT:4 * NSLOT]
  sem_o = sems[4 * NSLOT:4 * NSLOT + NIMG]

  pltpu.sync_copy(items_h.at[wid], idx_i)
  pltpu.sync_copy(cates_h.at[wid], idx_c)
  pltpu.sync_copy(wk_h.at[wid], idx_w)
  pltpu.sync_copy(hr_h.at[wid], idx_hr)
  pltpu.sync_copy(bh_h.at[wid], idx_b)
  pltpu.sync_copy(wkend_h.at[wid], sc_wkend)
  pltpu.sync_copy(days_h.at[wid], sc_days)
  pltpu.sync_copy(dte_h.at[wid], sc_dte)
  pltpu.sync_copy(wtab_h, wtab_v)
  pltpu.sync_copy(htab_h, htab_v)
  pltpu.sync_copy(btab_h, btab_v)

  lane = lax.iota(jnp.int32, 16)

  def fire_gathers(kk, b):
    pltpu.async_copy(itab_h.at[idx_i.at[kk]], item_b.at[b], sem_i[b])
    pltpu.async_copy(ctab_h.at[idx_c.at[kk]], cate_b.at[b], sem_c[b])

  def wait_gathers(kk, b):
    pltpu.make_async_copy(itab_h.at[idx_i.at[kk]], item_b.at[b],
                          sem_i[b]).wait()
    pltpu.make_async_copy(ctab_h.at[idx_c.at[kk]], cate_b.at[b],
                          sem_c[b]).wait()

  def assembles(kk, b, p):
    base = wid * PER_W + kk * CH
    yield item_b.at[b], stage.at[sid, p, :, pl.ds(0, ITEM_DIM)], sem_a[b]
    yield (cate_b.at[b], stage.at[sid, p, :, pl.ds(ITEM_DIM, CATE_DIM)],
           sem_a[b])
    yield (sm_b.at[b], out_h.at[pl.ds(base, CH), pl.ds(_SM_OFF, SMALL_D)],
           sem_s[b])

  def fire_assembles(kk, b, p):
    for src, dst, sem in assembles(kk, b, p):
      pltpu.async_copy(src, dst, sem)

  def wait_assembles(kk, b, p):
    for src, dst, sem in assembles(kk, b, p):
      pltpu.make_async_copy(src, dst, sem).wait()

  def writeout(kk, p):
    base = wid * PER_W + kk * CH
    return (stage.at[sid, p], out_h.at[pl.ds(base, CH),
                                       pl.ds(0, ITEM_DIM + CATE_DIM)])

  def fire_writeout(kk, p):
    src, dst = writeout(kk, p)
    pltpu.async_copy(src, dst, sem_o[p])

  def wait_writeout(kk, p):
    src, dst = writeout(kk, p)
    pltpu.make_async_copy(src, dst, sem_o[p]).wait()

  def smalldims(kk, b):
    def col(c):
      return jnp.full((16,), c, jnp.int32)

    for g in range(CH // 16):
      rows = g * 16 + lane
      wkv = idx_w[kk, pl.ds(g * 16, 16)] * WEEK_DIM
      hrv = idx_hr[kk, pl.ds(g * 16, 16)] * HOUR_DIM
      bhv = idx_b[kk, pl.ds(g * 16, 16)] * BEH_DIM
      ob = sm_b.at[b]
      for d in range(WEEK_DIM):
        plsc.store_scatter(ob, [rows, col(_W_OFF + d)],
                           plsc.load_gather(wtab_v, [wkv + d]))
      for d in range(HOUR_DIM):
        plsc.store_scatter(ob, [rows, col(_H_OFF + d)],
                           plsc.load_gather(htab_v, [hrv + d]))
      for d in range(BEH_DIM):
        plsc.store_scatter(ob, [rows, col(_B_OFF + d)],
                           plsc.load_gather(btab_v, [bhv + d]))
      plsc.store_scatter(ob, [rows, col(_S_OFF)],
                         sc_wkend[kk, pl.ds(g * 16, 16)])
      plsc.store_scatter(ob, [rows, col(_S_OFF + 1)],
                         sc_days[kk, pl.ds(g * 16, 16)])
      plsc.store_scatter(ob, [rows, col(_S_OFF + 2)],
                         sc_dte[kk, pl.ds(g * 16, 16)])

  def process(kk, b):
    nb = (b + 1) % NSLOT
    pb = (b - 1) % NSLOT
    p = b % NIMG

    @pl.when(kk + 1 < K)
    def _():
      fire_gathers(kk + 1, nb)
    smalldims(kk, b)

    @pl.when(kk >= 1)
    def _():
      wait_assembles(kk - 1, pb, 1 - p)
      fire_writeout(kk - 1, 1 - p)
    wait_gathers(kk, b)

    @pl.when(kk >= NIMG)
    def _():
      wait_writeout(kk - NIMG, p)
    fire_assembles(kk, b, p)

  fire_gathers(0, 0)

  def loop_body(i, carry):
    kk0 = NSLOT * i
    for b in range(NSLOT):
      process(kk0 + b, b)
    return carry

  lax.fori_loop(0, K // NSLOT, loop_body, None)

  wait_assembles(K - 1, (K - 1) % NSLOT, (K - 1) % NIMG)
  fire_writeout(K - 1, (K - 1) % NIMG)
  wait_writeout(K - 2, (K - 2) % NIMG)
  wait_writeout(K - 1, (K - 1) % NIMG)


@jax.jit
def _run(items3, cates3, wk3, hr3, bh3, wkend3, days3, dte3,
         item_table, cate_table, weekday_table, hour_table, behavior_table):
  mesh = plsc.VectorSubcoreMesh(core_axis_name="c", subcore_axis_name="s")
  kfn = functools.partial(
      pl.kernel,
      mesh=mesh,
      compiler_params=pltpu.CompilerParams(
          needs_layout_passes=False, use_tc_tiling_on_sc=False),
      out_type=jax.ShapeDtypeStruct((TOTAL, OUT_D), jnp.float32),
      scratch_types=[
          pltpu.VMEM((K, CH), jnp.int32),
          pltpu.VMEM((K, CH), jnp.int32),
          pltpu.VMEM((K, CH), jnp.int32),
          pltpu.VMEM((K, CH), jnp.int32),
          pltpu.VMEM((K, CH), jnp.int32),
          pltpu.VMEM((K, CH), jnp.float32),
          pltpu.VMEM((K, CH), jnp.float32),
          pltpu.VMEM((K, CH), jnp.float32),
          pltpu.VMEM((7 * WEEK_DIM,), jnp.float32),
          pltpu.VMEM((24 * HOUR_DIM,), jnp.float32),
          pltpu.VMEM((5 * BEH_DIM,), jnp.float32),
          pltpu.VMEM((NSLOT, CH, ITEM_DIM), jnp.float32),
          pltpu.VMEM((NSLOT, CH, CATE_DIM), jnp.float32),
          pltpu.VMEM((NSLOT, CH, SMALL_D), jnp.float32),
          pltpu.MemorySpace.VMEM_SHARED(
              (16, NIMG, CH, ITEM_DIM + CATE_DIM), jnp.float32),
      ] + [pltpu.SemaphoreType.DMA] * (4 * NSLOT + NIMG),
  )(_sc_body)
  return kfn(items3, cates3, wk3, hr3, bh3, wkend3, days3, dte3,
             item_table, cate_table, weekday_table.reshape(-1),
             hour_table.reshape(-1), behavior_table.reshape(-1))


def kernel(items, categories, weekdays, hours, behaviors, is_weekends,
           days_norm, days_to_end, item_table, cate_table, weekday_table,
           hour_table, behavior_table):
  shp3 = (NW, K, CH)
  out = _run(items.reshape(shp3), categories.reshape(shp3),
             weekdays.reshape(shp3), hours.reshape(shp3),
             behaviors.reshape(shp3), is_weekends.reshape(shp3),
             days_norm.reshape(shp3), days_to_end.reshape(shp3),
             item_table, cate_table, weekday_table, hour_table,
             behavior_table)
  return out.reshape(B, L, OUT_D)
